# baseline (speedup 1.0000x reference)
import functools

import jax
import jax.numpy as jnp
from jax import lax
from jax.experimental import pallas as pl
from jax.experimental.pallas import tpu as pltpu
from jax.experimental.pallas import tpu_sc as plsc

VOCAB_DIM = 1000000
MODEL_DIM = 64
BATCH = 16384
Q_MIN, Q_MAX = -128.0, 127.0
LANES = 16
CHUNKS = MODEL_DIM // LANES

_info = plsc.get_sparse_core_info()
NUM_CORES = _info.num_cores
NUM_SUBCORES = _info.num_subcores
NUM_WORKERS = NUM_CORES * NUM_SUBCORES
B_PER_W = BATCH // NUM_WORKERS

_ROUND_MAGIC = 12582912.0


def _round_nearest_even(v):
    m = jnp.float32(_ROUND_MAGIC)
    return (v + m) - m


_mesh = plsc.VectorSubcoreMesh(core_axis_name="c", subcore_axis_name="s")


@functools.partial(
    pl.kernel,
    mesh=_mesh,
    out_type=jax.ShapeDtypeStruct((BATCH, MODEL_DIM), jnp.float32),
    scratch_types=[
        pltpu.VMEM((B_PER_W,), jnp.int32),
        pltpu.VMEM((B_PER_W, MODEL_DIM), jnp.float32),
        pltpu.VMEM((B_PER_W,), jnp.float32),
        pltpu.SemaphoreType.DMA,
        pltpu.SemaphoreType.DMA,
    ],
    compiler_params=pltpu.CompilerParams(use_tc_tiling_on_sc=False),
)
def _embed_sc(x_hbm, w_hbm, s_hbm, out_hbm, idx_v, rows_v, sc_v, sem_w, sem_s):
    wid = lax.axis_index("s") * NUM_CORES + lax.axis_index("c")
    base = wid * B_PER_W
    pltpu.sync_copy(x_hbm.at[pl.ds(base, B_PER_W)], idx_v)
    cp_w = pltpu.async_copy(w_hbm.at[idx_v], rows_v, sem_w)
    cp_s = pltpu.async_copy(s_hbm.at[idx_v], sc_v, sem_s)
    cp_w.wait()
    cp_s.wait()

    def group_body(g, carry):
        sv = sc_v[pl.ds(g * LANES, LANES)]
        for i in range(LANES):
            r = g * LANES + i
            s = sv[i]
            for c in range(CHUNKS):
                v = rows_v[r, pl.ds(c * LANES, LANES)]
                q = jnp.minimum(jnp.maximum(_round_nearest_even(v), Q_MIN), Q_MAX)
                rows_v[r, pl.ds(c * LANES, LANES)] = q * s
        return carry

    lax.fori_loop(0, B_PER_W // LANES, group_body, 0)
    pltpu.sync_copy(rows_v, out_hbm.at[pl.ds(base, B_PER_W)])


def kernel(x, weights, scales):
    return _embed_sc(x.astype(jnp.int32), weights, scales)


# speedup vs baseline: 1.7223x; 1.7223x over previous
import functools

import jax
import jax.numpy as jnp
from jax import lax
from jax.experimental import pallas as pl
from jax.experimental.pallas import tpu as pltpu
from jax.experimental.pallas import tpu_sc as plsc

VOCAB_DIM = 1000000
MODEL_DIM = 64
BATCH = 16384
Q_MIN, Q_MAX = -128.0, 127.0
LANES = 16
CHUNKS = MODEL_DIM // LANES

_info = plsc.get_sparse_core_info()
NUM_CORES = _info.num_cores
NUM_SUBCORES = _info.num_subcores
NUM_WORKERS = NUM_CORES * NUM_SUBCORES
B_PER_W = BATCH // NUM_WORKERS
GROUPS = B_PER_W // LANES

_ROUND_MAGIC = 12582912.0


def _round_nearest_even(v):
    m = jnp.float32(_ROUND_MAGIC)
    return (v + m) - m


_mesh = plsc.VectorSubcoreMesh(core_axis_name="c", subcore_axis_name="s")


@functools.partial(
    pl.kernel,
    mesh=_mesh,
    out_type=jax.ShapeDtypeStruct((BATCH, MODEL_DIM), jnp.float32),
    scratch_types=[
        pltpu.VMEM((B_PER_W,), jnp.int32),
        pltpu.VMEM((B_PER_W, MODEL_DIM), jnp.float32),
        pltpu.VMEM((B_PER_W * 8,), jnp.float32),
        pltpu.SemaphoreType.DMA,
        pltpu.SemaphoreType.DMA,
    ],
    compiler_params=pltpu.CompilerParams(needs_layout_passes=False),
)
def _embed_sc(x_hbm, w_hbm, s_hbm, out_hbm, idx_v, rows_v, sc8_v, sem_w, sem_s):
    wid = lax.axis_index("s") * NUM_CORES + lax.axis_index("c")
    base = wid * B_PER_W
    pltpu.sync_copy(x_hbm.at[pl.ds(base, B_PER_W)], idx_v)

    def fire_body(g, carry):
        iv = idx_v[pl.ds(g * LANES, LANES)]
        iv_al = iv & jnp.int32(-8)
        for i in range(LANES):
            r = g * LANES + i
            pltpu.make_async_copy(w_hbm.at[iv[i]], rows_v.at[r], sem_w).start()
            pltpu.make_async_copy(
                s_hbm.at[pl.ds(pl.multiple_of(iv_al[i], 8), 8)],
                sc8_v.at[pl.ds(r * 8, 8)],
                sem_s,
            ).start()
        return carry

    lax.fori_loop(0, GROUPS, fire_body, 0)
    pltpu.make_async_copy(s_hbm.at[pl.ds(0, B_PER_W * 8)], sc8_v, sem_s).wait()
    pltpu.make_async_copy(w_hbm.at[pl.ds(0, B_PER_W)], rows_v, sem_w).wait()

    lane8 = jnp.arange(0, 8 * LANES, 8, dtype=jnp.int32)

    def group_body(g, carry):
        iv = idx_v[pl.ds(g * LANES, LANES)]
        pos = (g * (8 * LANES) + lane8) + (iv & jnp.int32(7))
        sv = plsc.load_gather(sc8_v, [pos])
        for i in range(LANES):
            r = g * LANES + i
            s = sv[i]
            for c in range(CHUNKS):
                v = rows_v[r, pl.ds(c * LANES, LANES)]
                q = jnp.minimum(jnp.maximum(_round_nearest_even(v), Q_MIN), Q_MAX)
                rows_v[r, pl.ds(c * LANES, LANES)] = q * s
        return carry

    lax.fori_loop(0, GROUPS, group_body, 0)
    pltpu.sync_copy(rows_v, out_hbm.at[pl.ds(base, B_PER_W)])


def kernel(x, weights, scales):
    return _embed_sc(x.astype(jnp.int32), weights, scales)
